# Initial kernel scaffold; baseline (speedup 1.0000x reference)
import jax
import jax.numpy as jnp
from jax.experimental import pallas as pl


def kernel(tokens, E0, E1, E2, E3, E4, E5, E6, E7, W, b):
    raise NotImplementedError("write your pallas kernel here")


# speedup vs baseline: 30.9263x; 30.9263x over previous
import jax
import jax.numpy as jnp
from jax import lax
from jax.experimental import pallas as pl
from jax.experimental.pallas import tpu as pltpu

B, S = 4096, 200
BS = B * S
TN = 2048

VOCABS = [7, 7, 2, 3, 4, 2, 10, 3]
WIDTHS = [16, 16, 8, 8, 16, 8, 16, 16]
WOFFS = [0, 16, 32, 40, 48, 64, 72, 88]
PAIRS = [(0, 1), (2, 3), (4, 5), (6, 7)]
PAIR_SIZES = [VOCABS[i] * VOCABS[j] for i, j in PAIRS]
PAIR_OFFS = [0, 49, 55, 63]
KTOT = 96


def _body(tok_ref, e0, e1, e2, e3, e4, e5, e6, e7, w_ref, b_ref, out_ref, tp_ref):
    es = [e0, e1, e2, e3, e4, e5, e6, e7]

    @pl.when(pl.program_id(0) == 0)
    def _build_pair_table():
        tp = jnp.zeros((KTOT, 128), jnp.float32)
        for g, (i, j) in enumerate(PAIRS):
            vi, vj = VOCABS[i], VOCABS[j]
            rg = vi * vj
            pi = jnp.dot(es[i][...], w_ref[WOFFS[i]:WOFFS[i] + WIDTHS[i], :],
                         preferred_element_type=jnp.float32)
            pj = jnp.dot(es[j][...], w_ref[WOFFS[j]:WOFFS[j] + WIDTHS[j], :],
                         preferred_element_type=jnp.float32)
            r_i = lax.broadcasted_iota(jnp.int32, (rg, vi), 0)
            c_i = lax.broadcasted_iota(jnp.int32, (rg, vi), 1)
            a0 = (r_i // vj == c_i).astype(jnp.float32)
            r_j = lax.broadcasted_iota(jnp.int32, (rg, vj), 0)
            c_j = lax.broadcasted_iota(jnp.int32, (rg, vj), 1)
            a1 = (r_j % vj == c_j).astype(jnp.float32)
            pp = (jnp.dot(a0, pi, preferred_element_type=jnp.float32)
                  + jnp.dot(a1, pj, preferred_element_type=jnp.float32))
            r_t = lax.broadcasted_iota(jnp.int32, (KTOT, rg), 0)
            c_t = lax.broadcasted_iota(jnp.int32, (KTOT, rg), 1)
            cg = (r_t - PAIR_OFFS[g] == c_t).astype(jnp.float32)
            tp = tp + jnp.dot(cg, pp, preferred_element_type=jnp.float32)
        r_b = lax.broadcasted_iota(jnp.int32, (KTOT, 1), 0)
        bcol = (r_b == 93).astype(jnp.float32)
        tp = tp + jnp.dot(bcol, b_ref[...], preferred_element_type=jnp.float32)
        tp_ref[...] = tp

    tok = tok_ref[...]
    lane = lax.broadcasted_iota(jnp.int32, (TN, KTOT), 1)
    mh = (lane == 93).astype(jnp.float32)
    for g, (i, j) in enumerate(PAIRS):
        pidx = tok[:, i:i + 1] * VOCABS[j] + tok[:, j:j + 1] + PAIR_OFFS[g]
        mh = mh + (lane == pidx).astype(jnp.float32)
    out_ref[...] = jnp.dot(mh, tp_ref[...], preferred_element_type=jnp.float32)


def kernel(tokens, E0, E1, E2, E3, E4, E5, E6, E7, W, b):
    tok2 = tokens.reshape(BS, 8)
    es = [E0, E1, E2, E3, E4, E5, E6, E7]
    grid = (BS // TN,)
    in_specs = [pl.BlockSpec((TN, 8), lambda i: (i, 0))]
    for t in range(8):
        v, w = VOCABS[t], WIDTHS[t]
        in_specs.append(pl.BlockSpec((v, w), lambda i: (0, 0)))
    in_specs.append(pl.BlockSpec((104, 128), lambda i: (0, 0)))
    in_specs.append(pl.BlockSpec((1, 128), lambda i: (0, 0)))
    out = pl.pallas_call(
        _body,
        grid=grid,
        in_specs=in_specs,
        out_specs=pl.BlockSpec((TN, 128), lambda i: (i, 0)),
        out_shape=jax.ShapeDtypeStruct((BS, 128), jnp.float32),
        scratch_shapes=[pltpu.VMEM((KTOT, 128), jnp.float32)],
        compiler_params=pltpu.CompilerParams(
            dimension_semantics=("arbitrary",)),
    )(tok2, *es, W, b.reshape(1, 128))
    return out.reshape(B, S, 128)
